# Initial kernel scaffold; baseline (speedup 1.0000x reference)
import jax
import jax.numpy as jnp
from jax.experimental import pallas as pl


def kernel(x, edge_index, batch, W1, b1, ln1_w, ln1_b, W2, b2, ln2_w, ln2_b, Wg1, bg1, Wg2, bg2, Wc, bc):
    raise NotImplementedError("write your pallas kernel here")


# speedup vs baseline: 6.1057x; 6.1057x over previous
import dataclasses
import functools

import jax
import jax.numpy as jnp
from jax import lax
from jax.experimental import pallas as pl
from jax.experimental.pallas import tpu as pltpu
from jax.experimental.pallas import tpu_sc as plsc

N = 10000
E = 320000
D = 128
H = 128
G = 64
C = 10

NP = 10240
NW = 32
FW = H // NW
ECHUNK = 2000
NCH = E // ECHUNK
EG = ECHUNK // 16
DSUB = 16
DEDGE = E // DSUB
DCH = DEDGE // ECHUNK
NSEG = NP // DSUB

RB = 1024
NB = NP // RB

_sc_mesh = plsc.VectorSubcoreMesh(core_axis_name="c", subcore_axis_name="s")

_cp = pltpu.CompilerParams()
if "needs_layout_passes" in pltpu.CompilerParams.__dataclass_fields__:
    _cp = dataclasses.replace(_cp, needs_layout_passes=False)


@functools.partial(
    pl.kernel,
    out_type=[jax.ShapeDtypeStruct((NP,), jnp.float32),
              jax.ShapeDtypeStruct((DSUB, NP), jnp.float32)],
    mesh=_sc_mesh,
    compiler_params=_cp,
    scratch_types=[
        pltpu.VMEM((ECHUNK,), jnp.int32),
        pltpu.VMEM((NP,), jnp.float32),
        pltpu.VMEM((DSUB, NSEG), jnp.float32),
        pltpu.VMEM((NSEG,), jnp.float32),
    ],
)
def _deg_kernel(dst_hbm, deg_hbm, part_hbm, dst_i, hist_v, comb_v, seg_v):
    c = lax.axis_index("c")
    s = lax.axis_index("s")
    ones16 = jnp.full((16,), 1.0, jnp.float32)

    @pl.when(c == 0)
    def _():
        @pl.loop(0, NP // 16)
        def _(i):
            hist_v[pl.ds(i * 16, 16)] = jnp.zeros((16,), jnp.float32)

        ebase = s * DEDGE

        @pl.loop(0, DCH)
        def _(j):
            pltpu.sync_copy(dst_hbm.at[pl.ds(ebase + j * ECHUNK, ECHUNK)],
                            dst_i)

            @pl.loop(0, EG)
            def _(g):
                dv = dst_i[pl.ds(g * 16, 16)]
                plsc.addupdate_scatter(hist_v, [dv], ones16)

        pltpu.sync_copy(hist_v, part_hbm.at[s])
        plsc.subcore_barrier()

        pltpu.sync_copy(part_hbm.at[:, pl.ds(s * NSEG, NSEG)], comb_v)

        @pl.loop(0, NSEG // 16)
        def _(v):
            acc = jnp.full((16,), 1.0, jnp.float32)
            for r in range(DSUB):
                acc = acc + comb_v[r, pl.ds(v * 16, 16)]
            seg_v[pl.ds(v * 16, 16)] = acc

        pltpu.sync_copy(seg_v, deg_hbm.at[pl.ds(s * NSEG, NSEG)])


@functools.partial(
    pl.kernel,
    out_type=jax.ShapeDtypeStruct((NW, FW, NP), jnp.float32),
    mesh=_sc_mesh,
    compiler_params=_cp,
    scratch_types=[
        pltpu.VMEM((ECHUNK,), jnp.int32),
        pltpu.VMEM((ECHUNK,), jnp.int32),
        pltpu.VMEM((FW, NP), jnp.float32),
        pltpu.VMEM((FW, NP), jnp.float32),
    ],
)
def _agg_kernel(hperm_hbm, src_hbm, dst_hbm, out_hbm,
                src_i, dst_i, t_v, acc_v):
    c = lax.axis_index("c")
    s = lax.axis_index("s")
    w = s * 2 + c
    kvecs = [jnp.full((16,), k, jnp.int32) for k in range(FW)]

    for k in range(FW):
        @pl.loop(0, NP // 16)
        def _(i):
            acc_v[k, pl.ds(i * 16, 16)] = jnp.zeros((16,), jnp.float32)

    pltpu.sync_copy(hperm_hbm.at[w], t_v)

    @pl.loop(0, NCH)
    def _(j):
        pltpu.sync_copy(src_hbm.at[pl.ds(j * ECHUNK, ECHUNK)], src_i)
        pltpu.sync_copy(dst_hbm.at[pl.ds(j * ECHUNK, ECHUNK)], dst_i)

        @pl.loop(0, EG)
        def _(g):
            sv = src_i[pl.ds(g * 16, 16)]
            dv = dst_i[pl.ds(g * 16, 16)]
            for k in range(FW):
                val = plsc.load_gather(t_v, [kvecs[k], sv])
                plsc.addupdate_scatter(acc_v, [kvecs[k], dv], val)

    pltpu.sync_copy(acc_v, out_hbm.at[w])


def _mm1_body(x_ref, w_ref, o_ref):
    o_ref[...] = jnp.dot(x_ref[...], w_ref[...],
                         preferred_element_type=jnp.float32)


def _mm1(x, W1):
    return pl.pallas_call(
        _mm1_body,
        grid=(NB,),
        in_specs=[pl.BlockSpec((RB, D), lambda i: (i, 0)),
                  pl.BlockSpec((D, H), lambda i: (0, 0))],
        out_specs=pl.BlockSpec((RB, H), lambda i: (i, 0)),
        out_shape=jax.ShapeDtypeStruct((NP, H), jnp.float32),
    )(x, W1)


def _store_perm(o_ref, hs):
    hsT = jnp.transpose(hs)
    for w in range(NW):
        o_ref[w] = hsT[FW * w:FW * (w + 1), :]


def _scale_split_body(h_ref, deg_ref, o_hp_ref, o_hs_ref, o_dis_ref):
    dis = lax.rsqrt(deg_ref[...])
    hs = h_ref[...] * dis
    o_dis_ref[...] = dis
    o_hs_ref[...] = hs
    _store_perm(o_hp_ref, hs)


def _scale_split(h, deg):
    return pl.pallas_call(
        _scale_split_body,
        grid=(NB,),
        in_specs=[pl.BlockSpec((RB, H), lambda i: (i, 0)),
                  pl.BlockSpec((RB, 1), lambda i: (i, 0))],
        out_specs=[pl.BlockSpec((NW, FW, RB), lambda i: (0, 0, i)),
                   pl.BlockSpec((RB, H), lambda i: (i, 0)),
                   pl.BlockSpec((RB, 1), lambda i: (i, 0))],
        out_shape=[jax.ShapeDtypeStruct((NW, FW, NP), jnp.float32),
                   jax.ShapeDtypeStruct((NP, H), jnp.float32),
                   jax.ShapeDtypeStruct((NP, 1), jnp.float32)],
    )(h, deg)


def _combine_body(accp_ref, hs_ref, dis_ref, b_ref, o_out_ref, o_part_ref):
    i = pl.program_id(0)
    accT = jnp.concatenate([accp_ref[w] for w in range(NW)], axis=0)
    acc = jnp.transpose(accT)
    out = dis_ref[...] * (acc + hs_ref[...]) + b_ref[...]
    o_out_ref[...] = out
    rowid = i * RB + lax.broadcasted_iota(jnp.int32, (RB, 1), 0)
    outm = jnp.where(rowid < N, out, 0.0)
    ssum = jnp.sum(outm)
    ssq = jnp.sum(outm * outm)
    lane = lax.broadcasted_iota(jnp.int32, (1, 1, H), 2)
    o_part_ref[...] = jnp.where(lane == 0, ssum,
                                jnp.where(lane == 1, ssq, 0.0))


def _combine(accp, hs, dis, b):
    return pl.pallas_call(
        _combine_body,
        grid=(NB,),
        in_specs=[pl.BlockSpec((NW, FW, RB), lambda i: (0, 0, i)),
                  pl.BlockSpec((RB, H), lambda i: (i, 0)),
                  pl.BlockSpec((RB, 1), lambda i: (i, 0)),
                  pl.BlockSpec((1, H), lambda i: (0, 0))],
        out_specs=[pl.BlockSpec((RB, H), lambda i: (i, 0)),
                   pl.BlockSpec((1, 1, H), lambda i: (i, 0, 0))],
        out_shape=[jax.ShapeDtypeStruct((NP, H), jnp.float32),
                   jax.ShapeDtypeStruct((NB, 1, H), jnp.float32)],
    )(accp, hs, dis, b)


def _ln_stats(part_ref):
    psum = jnp.sum(part_ref[:, :, 0:1])
    psq = jnp.sum(part_ref[:, :, 1:2])
    cnt = jnp.float32(N * H)
    mean = psum / cnt
    var = psq / cnt - mean * mean
    inv = lax.rsqrt(var + 1e-5)
    return mean, inv


def _mid_body(out1_ref, part_ref, lnw_ref, lnb_ref, w2_ref, dis_ref,
              o_hp_ref, o_hs_ref):
    mean, inv = _ln_stats(part_ref)
    h = (out1_ref[...] - mean) * inv * lnw_ref[...] + lnb_ref[...]
    h = jnp.maximum(h, 0.0)
    hw = jnp.dot(h, w2_ref[...], preferred_element_type=jnp.float32)
    hs = dis_ref[...] * hw
    o_hs_ref[...] = hs
    _store_perm(o_hp_ref, hs)


def _mid(out1, part, lnw, lnb, W2, dis):
    return pl.pallas_call(
        _mid_body,
        grid=(NB,),
        in_specs=[pl.BlockSpec((RB, H), lambda i: (i, 0)),
                  pl.BlockSpec((NB, 1, H), lambda i: (0, 0, 0)),
                  pl.BlockSpec((1, H), lambda i: (0, 0)),
                  pl.BlockSpec((1, H), lambda i: (0, 0)),
                  pl.BlockSpec((H, H), lambda i: (0, 0)),
                  pl.BlockSpec((RB, 1), lambda i: (i, 0))],
        out_specs=[pl.BlockSpec((NW, FW, RB), lambda i: (0, 0, i)),
                   pl.BlockSpec((RB, H), lambda i: (i, 0))],
        out_shape=[jax.ShapeDtypeStruct((NW, FW, NP), jnp.float32),
                   jax.ShapeDtypeStruct((NP, H), jnp.float32)],
    )(out1, part, lnw, lnb, W2, dis)


def _gate_body(out2_ref, part_ref, lnw_ref, lnb_ref, wg1_ref, bg1_ref,
               wg2_ref, bg2_ref, o_h_ref, o_gate_ref):
    i = pl.program_id(0)
    mean, inv = _ln_stats(part_ref)
    h = (out2_ref[...] - mean) * inv * lnw_ref[...] + lnb_ref[...]
    h = jnp.maximum(h, 0.0)
    o_h_ref[...] = h
    g1 = jnp.dot(h, wg1_ref[...], preferred_element_type=jnp.float32)
    g1 = jnp.maximum(g1 + bg1_ref[...], 0.0)
    gate = jnp.sum(g1 * wg2_ref[...], axis=1, keepdims=True) + bg2_ref[...]
    rowid = i * RB + lax.broadcasted_iota(jnp.int32, (RB, 1), 0)
    o_gate_ref[...] = jnp.where(rowid < N, gate, jnp.float32(-1e30))


def _gate(out2, part, lnw, lnb, Wg1, bg1, wg2row, bg2):
    return pl.pallas_call(
        _gate_body,
        grid=(NB,),
        in_specs=[pl.BlockSpec((RB, H), lambda i: (i, 0)),
                  pl.BlockSpec((NB, 1, H), lambda i: (0, 0, 0)),
                  pl.BlockSpec((1, H), lambda i: (0, 0)),
                  pl.BlockSpec((1, H), lambda i: (0, 0)),
                  pl.BlockSpec((H, G), lambda i: (0, 0)),
                  pl.BlockSpec((1, G), lambda i: (0, 0)),
                  pl.BlockSpec((1, G), lambda i: (0, 0)),
                  pl.BlockSpec((1, 1), lambda i: (0, 0))],
        out_specs=[pl.BlockSpec((RB, H), lambda i: (i, 0)),
                   pl.BlockSpec((RB, 1), lambda i: (i, 0))],
        out_shape=[jax.ShapeDtypeStruct((NP, H), jnp.float32),
                   jax.ShapeDtypeStruct((NP, 1), jnp.float32)],
    )(out2, part, lnw, lnb, Wg1, bg1, wg2row, bg2)


def _pool_body(h_ref, gate_ref, bcol_ref, brow_ref, wc_ref, bc_ref, o_ref):
    gate = gate_ref[...]
    oh = (bcol_ref[...] == lax.broadcasted_iota(jnp.int32, (1, G), 1))
    oh = oh.astype(jnp.float32)
    gmax = jnp.max(jnp.where(oh > 0, gate, jnp.float32(-1e30)),
                   axis=0, keepdims=True)
    gmax_row = jnp.sum(oh * gmax, axis=1, keepdims=True)
    eg = jnp.exp(gate - gmax_row)
    denom = jnp.sum(oh * eg, axis=0, keepdims=True)
    denom_row = jnp.sum(oh * denom, axis=1, keepdims=True)
    alpha = eg / (denom_row + 1e-16)
    ohT = (lax.broadcasted_iota(jnp.int32, (G, 1), 0) == brow_ref[0:1, :])
    ohT = ohT.astype(jnp.float32)
    pooled = jnp.dot(ohT, alpha * h_ref[...],
                     preferred_element_type=jnp.float32)
    o_ref[...] = jnp.dot(pooled, wc_ref[...],
                         preferred_element_type=jnp.float32) + bc_ref[...]


def _pool(h, gate, bcol, brow8, Wc, bc):
    return pl.pallas_call(
        _pool_body,
        out_shape=jax.ShapeDtypeStruct((G, C), jnp.float32),
    )(h, gate, bcol, brow8, Wc, bc)


def kernel(x, edge_index, batch, W1, b1, ln1_w, ln1_b, W2, b2, ln2_w, ln2_b,
           Wg1, bg1, Wg2, bg2, Wc, bc):
    src = edge_index[0]
    dst = edge_index[1]
    xp = jnp.concatenate([x, jnp.zeros((NP - N, D), jnp.float32)], axis=0)
    batchp = jnp.concatenate([batch, jnp.full((NP - N,), G, jnp.int32)])

    deg, _ = _deg_kernel(dst)
    h1 = _mm1(xp, W1)
    hperm1, hs1, dis = _scale_split(h1, deg.reshape(NP, 1))
    accp1 = _agg_kernel(hperm1, src, dst)
    out1, part1 = _combine(accp1, hs1, dis, b1.reshape(1, H))
    hperm2, hs2 = _mid(out1, part1, ln1_w.reshape(1, H), ln1_b.reshape(1, H),
                       W2, dis)
    accp2 = _agg_kernel(hperm2, src, dst)
    out2, part2 = _combine(accp2, hs2, dis, b2.reshape(1, H))
    h, gate = _gate(out2, part2, ln2_w.reshape(1, H), ln2_b.reshape(1, H),
                    Wg1, bg1.reshape(1, G), Wg2.reshape(1, G),
                    bg2.reshape(1, 1))
    bcol = batchp.reshape(NP, 1)
    brow8 = jnp.broadcast_to(batchp.reshape(1, NP), (8, NP))
    return _pool(h, gate, bcol, brow8, Wc, bc.reshape(1, C))


# speedup vs baseline: 17.9769x; 2.9443x over previous
import dataclasses
import functools

import jax
import jax.numpy as jnp
from jax import lax
from jax.experimental import pallas as pl
from jax.experimental.pallas import tpu as pltpu
from jax.experimental.pallas import tpu_sc as plsc

N = 10000
E = 320000
D = 128
H = 128
G = 64
C = 10

NP = 10240
NW = 32
FW = H // NW
ECHUNK = 3200
NCH = E // ECHUNK
EG = ECHUNK // 16
DSUB = 16
DEDGE = E // DSUB
DCHUNK = 2000
DCH = DEDGE // DCHUNK
NSEG = NP // DSUB

RB = 1024
NB = NP // RB

_sc_mesh = plsc.VectorSubcoreMesh(core_axis_name="c", subcore_axis_name="s")

_cp = pltpu.CompilerParams()
if "needs_layout_passes" in pltpu.CompilerParams.__dataclass_fields__:
    _cp = dataclasses.replace(_cp, needs_layout_passes=False)


@functools.partial(
    pl.kernel,
    out_type=[jax.ShapeDtypeStruct((NP,), jnp.float32),
              jax.ShapeDtypeStruct((DSUB, NP), jnp.float32)],
    mesh=_sc_mesh,
    compiler_params=_cp,
    scratch_types=[
        pltpu.VMEM((DCHUNK,), jnp.int32),
        pltpu.VMEM((NP,), jnp.float32),
        pltpu.VMEM((DSUB, NSEG), jnp.float32),
        pltpu.VMEM((NSEG,), jnp.float32),
    ],
)
def _deg_kernel(dst_hbm, deg_hbm, part_hbm, dst_i, hist_v, comb_v, seg_v):
    c = lax.axis_index("c")
    s = lax.axis_index("s")
    ones16 = jnp.full((16,), 1.0, jnp.float32)

    @pl.when(c == 0)
    def _():
        @pl.loop(0, NP // 16)
        def _(i):
            hist_v[pl.ds(i * 16, 16)] = jnp.zeros((16,), jnp.float32)

        ebase = s * DEDGE

        @pl.loop(0, DCH)
        def _(j):
            pltpu.sync_copy(dst_hbm.at[pl.ds(ebase + j * DCHUNK, DCHUNK)],
                            dst_i)

            @pl.loop(0, DCHUNK // 16)
            def _(g):
                dv = dst_i[pl.ds(g * 16, 16)]
                plsc.addupdate_scatter(hist_v, [dv], ones16)

        pltpu.sync_copy(hist_v, part_hbm.at[s])
        plsc.subcore_barrier()

        pltpu.sync_copy(part_hbm.at[:, pl.ds(s * NSEG, NSEG)], comb_v)

        @pl.loop(0, NSEG // 16)
        def _(v):
            acc = jnp.full((16,), 1.0, jnp.float32)
            for r in range(DSUB):
                acc = acc + comb_v[r, pl.ds(v * 16, 16)]
            seg_v[pl.ds(v * 16, 16)] = acc

        pltpu.sync_copy(seg_v, deg_hbm.at[pl.ds(s * NSEG, NSEG)])


@functools.partial(
    pl.kernel,
    out_type=jax.ShapeDtypeStruct((NW, FW, NP), jnp.float32),
    mesh=_sc_mesh,
    compiler_params=_cp,
    scratch_types=[
        pltpu.VMEM((2, ECHUNK), jnp.int32),
        pltpu.VMEM((2, ECHUNK), jnp.int32),
        pltpu.VMEM((FW, NP), jnp.float32),
        pltpu.VMEM((FW, NP), jnp.float32),
        pltpu.SemaphoreType.DMA,
        pltpu.SemaphoreType.DMA,
        pltpu.SemaphoreType.DMA,
        pltpu.SemaphoreType.DMA,
    ],
)
def _agg_kernel(hperm_hbm, src_hbm, dst_hbm, out_hbm,
                src_i, dst_i, t_v, acc_v, ss0, ss1, ds0, ds1):
    c = lax.axis_index("c")
    s = lax.axis_index("s")
    w = s * 2 + c
    kvecs = [jnp.full((16,), k, jnp.int32) for k in range(FW)]
    ssems = [ss0, ss1]
    dsems = [ds0, ds1]

    def idx_copies(jj, b):
        sl = pl.ds(jj * ECHUNK, ECHUNK)
        return (pltpu.make_async_copy(src_hbm.at[sl], src_i.at[b], ssems[b]),
                pltpu.make_async_copy(dst_hbm.at[sl], dst_i.at[b], dsems[b]))

    for cp in idx_copies(0, 0) + idx_copies(1, 1):
        cp.start()

    for k in range(FW):
        @plsc.parallel_loop(0, NP, step=16, unroll=8)
        def _(i):
            acc_v[k, pl.ds(i, 16)] = jnp.zeros((16,), jnp.float32)

    pltpu.sync_copy(hperm_hbm.at[w], t_v)

    @pl.loop(0, NCH, step=2)
    def _(j0):
        for b in range(2):
            jj = j0 + b
            for cp in idx_copies(jj, b):
                cp.wait()

            @plsc.parallel_loop(0, ECHUNK, step=16, unroll=4)
            def _(i):
                sv = src_i[b, pl.ds(i, 16)]
                dv = dst_i[b, pl.ds(i, 16)]
                for k in range(FW):
                    val = plsc.load_gather(t_v, [kvecs[k], sv])
                    plsc.addupdate_scatter(acc_v, [kvecs[k], dv], val)

            @pl.when(jj + 2 < NCH)
            def _():
                for cp in idx_copies(jj + 2, b):
                    cp.start()

    pltpu.sync_copy(acc_v, out_hbm.at[w])


def _mm1_body(x_ref, w_ref, o_ref):
    o_ref[...] = jnp.dot(x_ref[...], w_ref[...],
                         preferred_element_type=jnp.float32)


def _mm1(x, W1):
    return pl.pallas_call(
        _mm1_body,
        grid=(NB,),
        in_specs=[pl.BlockSpec((RB, D), lambda i: (i, 0)),
                  pl.BlockSpec((D, H), lambda i: (0, 0))],
        out_specs=pl.BlockSpec((RB, H), lambda i: (i, 0)),
        out_shape=jax.ShapeDtypeStruct((NP, H), jnp.float32),
    )(x, W1)


def _store_perm(o_ref, hs):
    hsT = jnp.transpose(hs)
    for w in range(NW):
        o_ref[w] = hsT[FW * w:FW * (w + 1), :]


def _scale_split_body(h_ref, deg_ref, o_hp_ref, o_hs_ref, o_dis_ref):
    dis = lax.rsqrt(deg_ref[...])
    hs = h_ref[...] * dis
    o_dis_ref[...] = dis
    o_hs_ref[...] = hs
    _store_perm(o_hp_ref, hs)


def _scale_split(h, deg):
    return pl.pallas_call(
        _scale_split_body,
        grid=(NB,),
        in_specs=[pl.BlockSpec((RB, H), lambda i: (i, 0)),
                  pl.BlockSpec((RB, 1), lambda i: (i, 0))],
        out_specs=[pl.BlockSpec((NW, FW, RB), lambda i: (0, 0, i)),
                   pl.BlockSpec((RB, H), lambda i: (i, 0)),
                   pl.BlockSpec((RB, 1), lambda i: (i, 0))],
        out_shape=[jax.ShapeDtypeStruct((NW, FW, NP), jnp.float32),
                   jax.ShapeDtypeStruct((NP, H), jnp.float32),
                   jax.ShapeDtypeStruct((NP, 1), jnp.float32)],
    )(h, deg)


def _combine_body(accp_ref, hs_ref, dis_ref, b_ref, o_out_ref, o_part_ref):
    i = pl.program_id(0)
    accT = jnp.concatenate([accp_ref[w] for w in range(NW)], axis=0)
    acc = jnp.transpose(accT)
    out = dis_ref[...] * (acc + hs_ref[...]) + b_ref[...]
    o_out_ref[...] = out
    rowid = i * RB + lax.broadcasted_iota(jnp.int32, (RB, 1), 0)
    outm = jnp.where(rowid < N, out, 0.0)
    ssum = jnp.sum(outm)
    ssq = jnp.sum(outm * outm)
    lane = lax.broadcasted_iota(jnp.int32, (1, 1, H), 2)
    o_part_ref[...] = jnp.where(lane == 0, ssum,
                                jnp.where(lane == 1, ssq, 0.0))


def _combine(accp, hs, dis, b):
    return pl.pallas_call(
        _combine_body,
        grid=(NB,),
        in_specs=[pl.BlockSpec((NW, FW, RB), lambda i: (0, 0, i)),
                  pl.BlockSpec((RB, H), lambda i: (i, 0)),
                  pl.BlockSpec((RB, 1), lambda i: (i, 0)),
                  pl.BlockSpec((1, H), lambda i: (0, 0))],
        out_specs=[pl.BlockSpec((RB, H), lambda i: (i, 0)),
                   pl.BlockSpec((1, 1, H), lambda i: (i, 0, 0))],
        out_shape=[jax.ShapeDtypeStruct((NP, H), jnp.float32),
                   jax.ShapeDtypeStruct((NB, 1, H), jnp.float32)],
    )(accp, hs, dis, b)


def _ln_stats(part_ref):
    psum = jnp.sum(part_ref[:, :, 0:1])
    psq = jnp.sum(part_ref[:, :, 1:2])
    cnt = jnp.float32(N * H)
    mean = psum / cnt
    var = psq / cnt - mean * mean
    inv = lax.rsqrt(var + 1e-5)
    return mean, inv


def _mid_body(out1_ref, part_ref, lnw_ref, lnb_ref, w2_ref, dis_ref,
              o_hp_ref, o_hs_ref):
    mean, inv = _ln_stats(part_ref)
    h = (out1_ref[...] - mean) * inv * lnw_ref[...] + lnb_ref[...]
    h = jnp.maximum(h, 0.0)
    hw = jnp.dot(h, w2_ref[...], preferred_element_type=jnp.float32)
    hs = dis_ref[...] * hw
    o_hs_ref[...] = hs
    _store_perm(o_hp_ref, hs)


def _mid(out1, part, lnw, lnb, W2, dis):
    return pl.pallas_call(
        _mid_body,
        grid=(NB,),
        in_specs=[pl.BlockSpec((RB, H), lambda i: (i, 0)),
                  pl.BlockSpec((NB, 1, H), lambda i: (0, 0, 0)),
                  pl.BlockSpec((1, H), lambda i: (0, 0)),
                  pl.BlockSpec((1, H), lambda i: (0, 0)),
                  pl.BlockSpec((H, H), lambda i: (0, 0)),
                  pl.BlockSpec((RB, 1), lambda i: (i, 0))],
        out_specs=[pl.BlockSpec((NW, FW, RB), lambda i: (0, 0, i)),
                   pl.BlockSpec((RB, H), lambda i: (i, 0))],
        out_shape=[jax.ShapeDtypeStruct((NW, FW, NP), jnp.float32),
                   jax.ShapeDtypeStruct((NP, H), jnp.float32)],
    )(out1, part, lnw, lnb, W2, dis)


def _gate_body(out2_ref, part_ref, lnw_ref, lnb_ref, wg1_ref, bg1_ref,
               wg2_ref, bg2_ref, o_h_ref, o_gate_ref):
    i = pl.program_id(0)
    mean, inv = _ln_stats(part_ref)
    h = (out2_ref[...] - mean) * inv * lnw_ref[...] + lnb_ref[...]
    h = jnp.maximum(h, 0.0)
    o_h_ref[...] = h
    g1 = jnp.dot(h, wg1_ref[...], preferred_element_type=jnp.float32)
    g1 = jnp.maximum(g1 + bg1_ref[...], 0.0)
    gate = jnp.sum(g1 * wg2_ref[...], axis=1, keepdims=True) + bg2_ref[...]
    rowid = i * RB + lax.broadcasted_iota(jnp.int32, (RB, 1), 0)
    o_gate_ref[...] = jnp.where(rowid < N, gate, jnp.float32(-1e30))


def _gate(out2, part, lnw, lnb, Wg1, bg1, wg2row, bg2):
    return pl.pallas_call(
        _gate_body,
        grid=(NB,),
        in_specs=[pl.BlockSpec((RB, H), lambda i: (i, 0)),
                  pl.BlockSpec((NB, 1, H), lambda i: (0, 0, 0)),
                  pl.BlockSpec((1, H), lambda i: (0, 0)),
                  pl.BlockSpec((1, H), lambda i: (0, 0)),
                  pl.BlockSpec((H, G), lambda i: (0, 0)),
                  pl.BlockSpec((1, G), lambda i: (0, 0)),
                  pl.BlockSpec((1, G), lambda i: (0, 0)),
                  pl.BlockSpec((1, 1), lambda i: (0, 0))],
        out_specs=[pl.BlockSpec((RB, H), lambda i: (i, 0)),
                   pl.BlockSpec((RB, 1), lambda i: (i, 0))],
        out_shape=[jax.ShapeDtypeStruct((NP, H), jnp.float32),
                   jax.ShapeDtypeStruct((NP, 1), jnp.float32)],
    )(out2, part, lnw, lnb, Wg1, bg1, wg2row, bg2)


def _pool_body(h_ref, gate_ref, bcol_ref, brow_ref, wc_ref, bc_ref, o_ref):
    gate = gate_ref[...]
    oh = (bcol_ref[...] == lax.broadcasted_iota(jnp.int32, (1, G), 1))
    oh = oh.astype(jnp.float32)
    gmax = jnp.max(jnp.where(oh > 0, gate, jnp.float32(-1e30)),
                   axis=0, keepdims=True)
    gmax_row = jnp.sum(oh * gmax, axis=1, keepdims=True)
    eg = jnp.exp(gate - gmax_row)
    denom = jnp.sum(oh * eg, axis=0, keepdims=True)
    denom_row = jnp.sum(oh * denom, axis=1, keepdims=True)
    alpha = eg / (denom_row + 1e-16)
    ohT = (lax.broadcasted_iota(jnp.int32, (G, 1), 0) == brow_ref[0:1, :])
    ohT = ohT.astype(jnp.float32)
    pooled = jnp.dot(ohT, alpha * h_ref[...],
                     preferred_element_type=jnp.float32)
    o_ref[...] = jnp.dot(pooled, wc_ref[...],
                         preferred_element_type=jnp.float32) + bc_ref[...]


def _pool(h, gate, bcol, brow8, Wc, bc):
    return pl.pallas_call(
        _pool_body,
        out_shape=jax.ShapeDtypeStruct((G, C), jnp.float32),
    )(h, gate, bcol, brow8, Wc, bc)


def kernel(x, edge_index, batch, W1, b1, ln1_w, ln1_b, W2, b2, ln2_w, ln2_b,
           Wg1, bg1, Wg2, bg2, Wc, bc):
    src = edge_index[0]
    dst = edge_index[1]
    xp = jnp.concatenate([x, jnp.zeros((NP - N, D), jnp.float32)], axis=0)
    batchp = jnp.concatenate([batch, jnp.full((NP - N,), G, jnp.int32)])

    deg, _ = _deg_kernel(dst)
    h1 = _mm1(xp, W1)
    hperm1, hs1, dis = _scale_split(h1, deg.reshape(NP, 1))
    accp1 = _agg_kernel(hperm1, src, dst)
    out1, part1 = _combine(accp1, hs1, dis, b1.reshape(1, H))
    hperm2, hs2 = _mid(out1, part1, ln1_w.reshape(1, H), ln1_b.reshape(1, H),
                       W2, dis)
    accp2 = _agg_kernel(hperm2, src, dst)
    out2, part2 = _combine(accp2, hs2, dis, b2.reshape(1, H))
    h, gate = _gate(out2, part2, ln2_w.reshape(1, H), ln2_b.reshape(1, H),
                    Wg1, bg1.reshape(1, G), Wg2.reshape(1, G),
                    bg2.reshape(1, 1))
    bcol = batchp.reshape(NP, 1)
    brow8 = jnp.broadcast_to(batchp.reshape(1, NP), (8, NP))
    return _pool(h, gate, bcol, brow8, Wc, bc.reshape(1, C))
